# baseline (speedup 1.0000x reference)
import functools

import jax
import jax.numpy as jnp
from jax.experimental import pallas as pl


def _order_body(query_ref, text_ref, out_ref, *, q, t):
    s = jax.lax.dot_general(
        query_ref[0], text_ref[0], (((1,), (1,)), ((), ())))
    colmax = jnp.max(s, axis=0, keepdims=True)
    qi = jax.lax.broadcasted_iota(jnp.int32, (q, t), 0)
    minq = jnp.min(jnp.where(s == colmax, qi, q), axis=0, keepdims=True)
    first = (qi == minq).astype(jnp.float32)

    ji = jax.lax.broadcasted_iota(jnp.int32, (t, 2), 0)
    ci = jax.lax.broadcasted_iota(jnp.int32, (t, 2), 1)
    rhs = jnp.where(ci == 0, (t - ji).astype(jnp.float32), 1.0)
    sums = jax.lax.dot_general(
        first, rhs, (((1,), (0,)), ((), ())),
        precision=jax.lax.Precision.HIGHEST)
    wsum = sums[:, 0:1]
    cnt = sums[:, 1:2]
    o_col = q - wsum / (cnt + 0.001)

    ii = jax.lax.broadcasted_iota(jnp.int32, (q, q), 0)
    jj = jax.lax.broadcasted_iota(jnp.int32, (q, q), 1)
    diag_o = jnp.where(ii == jj, jnp.broadcast_to(o_col, (q, q)), 0.0)
    ones = jnp.full((q, q), 1.0, dtype=jnp.float32)
    C = jax.lax.dot_general(
        ones, diag_o, (((1,), (0,)), ((), ())),
        precision=jax.lax.Precision.HIGHEST)
    lt = C < o_col
    eqo = (C == o_col) & (jj < ii)
    rank = jnp.sum((lt | eqo).astype(jnp.int32), axis=1, keepdims=True)
    sel = rank == jj
    out_ref[0] = jnp.sum(jnp.where(sel, ii, 0), axis=0, keepdims=True)


def kernel(query, text):
    b, q, d = query.shape
    t = text.shape[1]
    out = pl.pallas_call(
        functools.partial(_order_body, q=q, t=t),
        grid=(b,),
        in_specs=[
            pl.BlockSpec((1, q, d), lambda i: (i, 0, 0)),
            pl.BlockSpec((1, t, d), lambda i: (i, 0, 0)),
        ],
        out_specs=pl.BlockSpec((1, 1, q), lambda i: (i, 0, 0)),
        out_shape=jax.ShapeDtypeStruct((b, 1, q), jnp.int32),
    )(query, text)
    return out.reshape(b, q)


# speedup vs baseline: 2.2827x; 2.2827x over previous
import functools

import jax
import jax.numpy as jnp
from jax.experimental import pallas as pl


def _order_body(text_ref, query_ref, out_ref, *, q, t):
    s = jax.lax.dot_general(
        text_ref[0], query_ref[0], (((1,), (1,)), ((), ())))
    rowmax = jnp.max(s, axis=1, keepdims=True)
    eqb = s >= rowmax
    eq_f = eqb.astype(jnp.float32)
    ii = jax.lax.broadcasted_iota(jnp.int32, (q, q), 0)
    jj = jax.lax.broadcasted_iota(jnp.int32, (q, q), 1)
    tstrict = (ii < jj).astype(jnp.float32)
    excl = jax.lax.dot_general(
        eq_f, tstrict, (((1,), (0,)), ((), ())))
    first = jnp.where(eqb & (excl == 0.0), 1.0, 0.0)
    rowi = jax.lax.broadcasted_iota(jnp.int32, (t, q), 0)
    wrow = (t - rowi).astype(jnp.float32)
    wsum = jnp.sum(first * wrow, axis=0, keepdims=True)
    cnt = jnp.sum(first, axis=0, keepdims=True)
    o_row = q - wsum / (cnt + 0.001)

    C = jnp.broadcast_to(o_row, (q, q))
    ii = jax.lax.broadcasted_iota(jnp.int32, (q, q), 0)
    jj = jax.lax.broadcasted_iota(jnp.int32, (q, q), 1)
    diag = (ii == jj).astype(jnp.float32)
    o_col = jnp.sum(C * diag, axis=1, keepdims=True)
    lt = C < o_col
    eqo = (C == o_col) & (jj < ii)
    rank = jnp.sum((lt | eqo).astype(jnp.int32), axis=1, keepdims=True)
    sel = rank == jj
    out_ref[0] = jnp.sum(jnp.where(sel, ii, 0), axis=0, keepdims=True)


def kernel(query, text):
    b, q, d = query.shape
    t = text.shape[1]
    out = pl.pallas_call(
        functools.partial(_order_body, q=q, t=t),
        grid=(b,),
        in_specs=[
            pl.BlockSpec((1, t, d), lambda i: (i, 0, 0)),
            pl.BlockSpec((1, q, d), lambda i: (i, 0, 0)),
        ],
        out_specs=pl.BlockSpec((1, 1, q), lambda i: (i, 0, 0)),
        out_shape=jax.ShapeDtypeStruct((b, 1, q), jnp.int32),
    )(text, query)
    return out.reshape(b, q)
